# baseline (speedup 1.0000x reference)
import jax
import jax.numpy as jnp
from jax import lax
from jax.experimental import pallas as pl
from jax.experimental.pallas import tpu as pltpu
from jax.experimental.pallas import tpu_sc as plsc

B, N, F = 1024, 200, 8
NUM_TYPES, EMBED_DIM = 8, 256

NC, NS, L = 2, 16, 16
NW = NC * NS
ROWS = B * N
PER_W = ROWS // NW
CHUNK = 128
N_CHUNKS = PER_W // CHUNK
NBUF = 2
REP = 2048


def _sc_body(in_hbm, tab_hbm, out_hbm, in_v, idx_v, rows_v, gsems, osems):
    wid = lax.axis_index("s") * NC + lax.axis_index("c")
    base = wid * PER_W

    pltpu.sync_copy(in_hbm.at[pl.ds(base * F, PER_W * F)], in_v)

    strided = lax.iota(jnp.int32, L) * F + 2
    rep_base = lax.iota(jnp.int32, L) * NUM_TYPES

    def build_idx(c):
        for j in range(CHUNK // L):
            vals = plsc.load_gather(
                in_v, [strided + (c * (CHUNK * F) + j * (L * F))])
            rep_off = rep_base + (
                ((c * (CHUNK // L) + j) % (REP // L)) * (L * NUM_TYPES))
            idx_v[c, pl.ds(j * L, L)] = rep_off + jnp.clip(
                vals.astype(jnp.int32), 0, NUM_TYPES - 1)

    def start_gather(c, b):
        pltpu.async_copy(tab_hbm.at[idx_v.at[c]], rows_v[b], gsems[b])

    for b in range(NBUF):
        build_idx(b)
        start_gather(b, b)

    def group_body(i, carry):
        for b in range(NBUF):
            c = i * NBUF + b

            @pl.when(c + NBUF < N_CHUNKS)
            def _():
                build_idx(c + NBUF)

            pltpu.make_async_copy(
                tab_hbm.at[idx_v.at[c]], rows_v[b], gsems[b]).wait()
            pltpu.async_copy(
                rows_v[b], out_hbm.at[pl.ds(base + c * CHUNK, CHUNK)],
                osems[b])

            @pl.when(c + NBUF < N_CHUNKS)
            def _():
                pltpu.make_async_copy(
                    rows_v[b],
                    out_hbm.at[pl.ds(base + c * CHUNK, CHUNK)],
                    osems[b]).wait()
                start_gather(c + NBUF, b)
        return carry

    lax.fori_loop(0, N_CHUNKS // NBUF, group_body, 0)

    for b in range(NBUF):
        c = N_CHUNKS - NBUF + b
        pltpu.make_async_copy(
            rows_v[b],
            out_hbm.at[pl.ds(base + c * CHUNK, CHUNK)],
            osems[b]).wait()


@jax.jit
def _sc_lookup(flat_inputs, rep_tab):
    mesh = plsc.VectorSubcoreMesh(
        core_axis_name="c", subcore_axis_name="s",
        num_cores=NC, num_subcores=NS,
    )
    return pl.kernel(
        _sc_body,
        out_type=jax.ShapeDtypeStruct((ROWS, EMBED_DIM), jnp.float32),
        mesh=mesh,
        scratch_types=[
            pltpu.VMEM((PER_W * F,), jnp.float32),
            pltpu.VMEM((N_CHUNKS, CHUNK), jnp.int32),
            [pltpu.VMEM((CHUNK, EMBED_DIM), jnp.float32)] * NBUF,
            [pltpu.SemaphoreType.DMA] * NBUF,
            [pltpu.SemaphoreType.DMA] * NBUF,
        ],
        compiler_params=pltpu.CompilerParams(needs_layout_passes=False),
    )(flat_inputs, rep_tab)


def kernel(inputs, type_embed):
    rep_tab = jnp.tile(type_embed, (REP, 1))
    out = _sc_lookup(inputs.reshape(ROWS * F), rep_tab)
    return out.reshape(B, N, EMBED_DIM)


# speedup vs baseline: 1.0147x; 1.0147x over previous
import jax
import jax.numpy as jnp
from jax import lax
from jax.experimental import pallas as pl
from jax.experimental.pallas import tpu as pltpu
from jax.experimental.pallas import tpu_sc as plsc

B, N, F = 1024, 200, 8
NUM_TYPES, EMBED_DIM = 8, 256

NC, NS, L = 2, 16, 16
NW = NC * NS
ROWS = B * N
PER_W = ROWS // NW
CHUNK = 128
N_CHUNKS = PER_W // CHUNK
NBUF = 2
REP = 512


def _sc_body(in_hbm, tab_hbm, out_hbm, in_v, idx_v, rows_v, gsems, osems):
    wid = lax.axis_index("s") * NC + lax.axis_index("c")
    base = wid * PER_W

    pltpu.sync_copy(in_hbm.at[pl.ds(base * F, PER_W * F)], in_v)

    strided = lax.iota(jnp.int32, L) * F + 2
    rep_base = lax.iota(jnp.int32, L) * NUM_TYPES
    phase = wid * (REP // NW)

    def build_idx(c):
        for j in range(CHUNK // L):
            vals = plsc.load_gather(
                in_v, [strided + (c * (CHUNK * F) + j * (L * F))])
            rep_off = rep_base + (
                ((phase + c * (CHUNK // L) + j) % (REP // L))
                * (L * NUM_TYPES))
            idx_v[c, pl.ds(j * L, L)] = rep_off + jnp.clip(
                vals.astype(jnp.int32), 0, NUM_TYPES - 1)

    def start_gather(c, b):
        pltpu.async_copy(tab_hbm.at[idx_v.at[c]], rows_v[b], gsems[b])

    for b in range(NBUF):
        build_idx(b)
        start_gather(b, b)

    def group_body(i, carry):
        for b in range(NBUF):
            c = i * NBUF + b

            @pl.when(c + NBUF < N_CHUNKS)
            def _():
                build_idx(c + NBUF)

            pltpu.make_async_copy(
                tab_hbm.at[idx_v.at[c]], rows_v[b], gsems[b]).wait()
            pltpu.async_copy(
                rows_v[b], out_hbm.at[pl.ds(base + c * CHUNK, CHUNK)],
                osems[b])

            @pl.when(c + NBUF < N_CHUNKS)
            def _():
                pltpu.make_async_copy(
                    rows_v[b],
                    out_hbm.at[pl.ds(base + c * CHUNK, CHUNK)],
                    osems[b]).wait()
                start_gather(c + NBUF, b)
        return carry

    lax.fori_loop(0, N_CHUNKS // NBUF, group_body, 0)

    for b in range(NBUF):
        c = N_CHUNKS - NBUF + b
        pltpu.make_async_copy(
            rows_v[b],
            out_hbm.at[pl.ds(base + c * CHUNK, CHUNK)],
            osems[b]).wait()


@jax.jit
def _sc_lookup(flat_inputs, rep_tab):
    mesh = plsc.VectorSubcoreMesh(
        core_axis_name="c", subcore_axis_name="s",
        num_cores=NC, num_subcores=NS,
    )
    return pl.kernel(
        _sc_body,
        out_type=jax.ShapeDtypeStruct((ROWS, EMBED_DIM), jnp.float32),
        mesh=mesh,
        scratch_types=[
            pltpu.VMEM((PER_W * F,), jnp.float32),
            pltpu.VMEM((N_CHUNKS, CHUNK), jnp.int32),
            [pltpu.VMEM((CHUNK, EMBED_DIM), jnp.float32)] * NBUF,
            [pltpu.SemaphoreType.DMA] * NBUF,
            [pltpu.SemaphoreType.DMA] * NBUF,
        ],
        compiler_params=pltpu.CompilerParams(needs_layout_passes=False),
    )(flat_inputs, rep_tab)


def kernel(inputs, type_embed):
    rep_tab = jnp.tile(type_embed, (REP, 1))
    out = _sc_lookup(inputs.reshape(ROWS * F), rep_tab)
    return out.reshape(B, N, EMBED_DIM)
